# baseline (speedup 1.0000x reference)
import functools
import jax
import jax.numpy as jnp
from jax import lax
from jax.experimental import pallas as pl
from jax.experimental.pallas import tpu as pltpu
from jax.experimental.pallas import tpu_sc as plsc

BATCH = 16384
INPUT_SIZE = 26
EMBED_DIM = 16
VOCAB = 1000000

NUM_WORKERS = 32
ROWS_PER_WORKER = BATCH // NUM_WORKERS
CHUNK_ROWS = 128
NUM_CHUNKS = ROWS_PER_WORKER // CHUNK_ROWS

LANE_TILES = (VOCAB + 127) // 128
TILES_PER_WORKER = (LANE_TILES + NUM_WORKERS - 1) // NUM_WORKERS
LAST_TILE_COLS = VOCAB - (LANE_TILES - 1) * 128


def _make_detile():
    info = plsc.get_sparse_core_info()
    nc = info.num_cores
    mesh = plsc.VectorSubcoreMesh(core_axis_name="c", subcore_axis_name="s")

    @functools.partial(
        pl.kernel,
        out_type=(
            jax.ShapeDtypeStruct((INPUT_SIZE * BATCH,), jnp.int32),
            jax.ShapeDtypeStruct((VOCAB * EMBED_DIM,), jnp.float32),
        ),
        mesh=mesh,
        scratch_types=[
            pltpu.VMEM((8, ROWS_PER_WORKER), jnp.int32),
            pltpu.VMEM((EMBED_DIM, 128), jnp.float32),
            pltpu.VMEM((128 * EMBED_DIM,), jnp.float32),
        ],
        compiler_params=pltpu.CompilerParams(
            needs_layout_passes=False, use_tc_tiling_on_sc=True),
    )
    def detile(xt_hbm, tt_hbm, tail_hbm, xout_hbm, tout_hbm, xv, v, o):
        wid = lax.axis_index("s") * nc + lax.axis_index("c")
        col0 = wid * ROWS_PER_WORKER

        @pl.when(wid == 0)
        def _tail():
            pltpu.sync_copy(tail_hbm, o.at[pl.ds(0, LAST_TILE_COLS *
                                                 EMBED_DIM)])
            pltpu.sync_copy(
                o.at[pl.ds(0, LAST_TILE_COLS * EMBED_DIM)],
                tout_hbm.at[pl.ds((LANE_TILES - 1) * 128 * EMBED_DIM,
                                  LAST_TILE_COLS * EMBED_DIM)])

        for t in range((INPUT_SIZE + 7) // 8):
            nr = min(8, INPUT_SIZE - t * 8)
            pltpu.sync_copy(
                xt_hbm.at[pl.ds(t * 8, nr), pl.ds(col0, ROWS_PER_WORKER)],
                xv.at[pl.ds(0, nr)])
            for r in range(nr):
                j = t * 8 + r
                pltpu.sync_copy(
                    xv.at[r],
                    xout_hbm.at[pl.ds(j * BATCH + col0, ROWS_PER_WORKER)])

        lane16 = lax.iota(jnp.int32, EMBED_DIM)

        def tile_body(k, _):
            t = wid + k * NUM_WORKERS

            @pl.when(t < LANE_TILES - 1)
            def _full():
                pltpu.sync_copy(tt_hbm.at[:, pl.ds(t * 128, 128)], v)

                def col_group(g, _):
                    for u in range(8):
                        c = g * 8 + u
                        row = plsc.load_gather(
                            v, [lane16, jnp.full((EMBED_DIM,), c, jnp.int32)])
                        o[pl.ds(c * EMBED_DIM, EMBED_DIM)] = row
                    return 0

                lax.fori_loop(0, 16, col_group, 0)
                pltpu.sync_copy(
                    o, tout_hbm.at[pl.ds(t * 128 * EMBED_DIM,
                                         128 * EMBED_DIM)])

            return 0

        lax.fori_loop(0, TILES_PER_WORKER, tile_body, 0)

    return detile


def _make_kernel():
    info = plsc.get_sparse_core_info()
    nc = info.num_cores
    mesh = plsc.VectorSubcoreMesh(core_axis_name="c", subcore_axis_name="s")

    @functools.partial(
        pl.kernel,
        out_type=jax.ShapeDtypeStruct((BATCH,), jnp.float32),
        mesh=mesh,
        scratch_types=[
            pltpu.VMEM((INPUT_SIZE, ROWS_PER_WORKER), jnp.int32),
            pltpu.VMEM((INPUT_SIZE, CHUNK_ROWS, EMBED_DIM), jnp.float32),
            pltpu.VMEM((INPUT_SIZE, CHUNK_ROWS, EMBED_DIM), jnp.float32),
            pltpu.VMEM((INPUT_SIZE * EMBED_DIM,), jnp.float32),
            pltpu.VMEM((EMBED_DIM,), jnp.float32),
            pltpu.VMEM((ROWS_PER_WORKER,), jnp.float32),
            pltpu.SemaphoreType.DMA,
            pltpu.SemaphoreType.DMA,
        ],
        compiler_params=pltpu.CompilerParams(
            needs_layout_passes=False, use_tc_tiling_on_sc=False),
    )
    def emb_kernel(x1_hbm, w_hbm, b_hbm, table_hbm, y_hbm,
                   idx_v, rows_a, rows_b, w_v, b_v, y_v, sem_a, sem_b):
        wid = lax.axis_index("s") * nc + lax.axis_index("c")
        row0 = wid * ROWS_PER_WORKER

        idx_descs = [
            pltpu.async_copy(
                x1_hbm.at[pl.ds(j * BATCH + row0, ROWS_PER_WORKER)],
                idx_v.at[j],
                sem_a,
            )
            for j in range(INPUT_SIZE)
        ]
        pltpu.sync_copy(w_hbm, w_v)
        pltpu.sync_copy(b_hbm, b_v)
        for d in idx_descs:
            d.wait()

        def gather(buf, c, sem):
            descs = []
            for j in range(INPUT_SIZE):
                descs.append(pltpu.async_copy(
                    table_hbm.at[idx_v.at[j, pl.ds(c * CHUNK_ROWS,
                                                   CHUNK_ROWS)]],
                    buf.at[j],
                    sem,
                ))
            return descs

        lane = lax.iota(jnp.int32, EMBED_DIM)

        def compute(buf, c):
            bias = b_v[:][0]

            def group_body(g, _):
                def row_body(rr, yvec):
                    r = g * 16 + rr
                    acc = buf[0, r, :] * w_v[pl.ds(0, EMBED_DIM)]
                    for j in range(1, INPUT_SIZE):
                        acc = (acc +
                               buf[j, r, :] * w_v[pl.ds(j * EMBED_DIM,
                                                        EMBED_DIM)])
                    val = jnp.sum(acc) + bias
                    return jnp.where(lane == rr, val, yvec)

                yvec = lax.fori_loop(
                    0, 16, row_body, jnp.zeros((EMBED_DIM,), jnp.float32))
                y_v[pl.ds(c * CHUNK_ROWS + g * 16, 16)] = yvec
                return 0

            lax.fori_loop(0, CHUNK_ROWS // 16, group_body, 0)

        bufs = (rows_a, rows_b)
        sems = (sem_a, sem_b)
        pending = gather(bufs[0], 0, sems[0])
        for c in range(NUM_CHUNKS):
            for d in pending:
                d.wait()
            if c + 1 < NUM_CHUNKS:
                pending = gather(bufs[(c + 1) % 2], c + 1, sems[(c + 1) % 2])
            compute(bufs[c % 2], c)

        pltpu.sync_copy(y_v, y_hbm.at[pl.ds(row0, ROWS_PER_WORKER)])

    return emb_kernel


_DETILE = _make_detile()
_EMB_KERNEL = _make_kernel()


@jax.jit
def kernel(X, table, W, b):
    tail = table[VOCAB - LAST_TILE_COLS:, :].reshape(-1)
    x1, tlin = _DETILE(X.T.astype(jnp.int32), table.T, tail)
    b16 = jnp.broadcast_to(b, (EMBED_DIM,)).astype(jnp.float32)
    y = _EMB_KERNEL(x1, W.reshape(-1), b16,
                    tlin.reshape(VOCAB, EMBED_DIM))
    return y.reshape(BATCH, 1)


# speedup vs baseline: 1.4415x; 1.4415x over previous
import functools
import jax
import jax.numpy as jnp
from jax import lax
from jax.experimental import pallas as pl
from jax.experimental.pallas import tpu as pltpu
from jax.experimental.pallas import tpu_sc as plsc

BATCH = 16384
INPUT_SIZE = 26
EMBED_DIM = 16
VOCAB = 1000000

NUM_WORKERS = 32
ROWS_PER_WORKER = BATCH // NUM_WORKERS
CHUNK_ROWS = 128
NUM_CHUNKS = ROWS_PER_WORKER // CHUNK_ROWS

LANE_TILES = (VOCAB + 127) // 128
TILES_PER_WORKER = (LANE_TILES + NUM_WORKERS - 1) // NUM_WORKERS
LAST_TILE_COLS = VOCAB - (LANE_TILES - 1) * 128


def _make_detile():
    info = plsc.get_sparse_core_info()
    nc = info.num_cores
    mesh = plsc.VectorSubcoreMesh(core_axis_name="c", subcore_axis_name="s")

    @functools.partial(
        pl.kernel,
        out_type=(
            jax.ShapeDtypeStruct((INPUT_SIZE * BATCH,), jnp.int32),
            jax.ShapeDtypeStruct((VOCAB * EMBED_DIM,), jnp.float32),
        ),
        mesh=mesh,
        scratch_types=[
            pltpu.VMEM((8, ROWS_PER_WORKER), jnp.int32),
            pltpu.VMEM((EMBED_DIM, 128), jnp.float32),
            pltpu.VMEM((EMBED_DIM, 128), jnp.float32),
            pltpu.VMEM((128 * EMBED_DIM,), jnp.float32),
            pltpu.VMEM((128 * EMBED_DIM,), jnp.float32),
            pltpu.SemaphoreType.DMA,
            pltpu.SemaphoreType.DMA,
            pltpu.SemaphoreType.DMA,
            pltpu.SemaphoreType.DMA,
        ],
        compiler_params=pltpu.CompilerParams(
            needs_layout_passes=False, use_tc_tiling_on_sc=True),
    )
    def detile(xt_hbm, tt_hbm, tail_hbm, xout_hbm, tout_hbm,
               xv, va, vb, oa, ob, sia, sib, soa, sob):
        wid = lax.axis_index("s") * nc + lax.axis_index("c")
        col0 = wid * ROWS_PER_WORKER

        @pl.when(wid == 0)
        def _tail():
            pltpu.sync_copy(tail_hbm, oa.at[pl.ds(0, LAST_TILE_COLS *
                                                  EMBED_DIM)])
            pltpu.sync_copy(
                oa.at[pl.ds(0, LAST_TILE_COLS * EMBED_DIM)],
                tout_hbm.at[pl.ds((LANE_TILES - 1) * 128 * EMBED_DIM,
                                  LAST_TILE_COLS * EMBED_DIM)])

        for t in range((INPUT_SIZE + 7) // 8):
            nr = min(8, INPUT_SIZE - t * 8)
            pltpu.sync_copy(
                xt_hbm.at[pl.ds(t * 8, nr), pl.ds(col0, ROWS_PER_WORKER)],
                xv.at[pl.ds(0, nr)])
            for r in range(nr):
                j = t * 8 + r
                pltpu.sync_copy(
                    xv.at[r],
                    xout_hbm.at[pl.ds(j * BATCH + col0, ROWS_PER_WORKER)])

        lane16 = lax.iota(jnp.int32, EMBED_DIM)
        nt = LANE_TILES - 1

        def start_in(v, sem, t):
            pltpu.async_copy(tt_hbm.at[:, pl.ds(t * 128, 128)], v, sem)

        def wait_in(v, sem, t):
            pltpu.make_async_copy(
                tt_hbm.at[:, pl.ds(t * 128, 128)], v, sem).wait()

        def start_out(o, sem, t):
            pltpu.async_copy(
                o, tout_hbm.at[pl.ds(t * 128 * EMBED_DIM,
                                     128 * EMBED_DIM)], sem)

        def wait_out(o, sem, t):
            pltpu.make_async_copy(
                o, tout_hbm.at[pl.ds(t * 128 * EMBED_DIM,
                                     128 * EMBED_DIM)], sem).wait()

        def transpose(v, o):
            for u in range(128):
                row = plsc.load_gather(
                    v, [lane16, jnp.full((EMBED_DIM,), u, jnp.int32)])
                o[pl.ds(u * EMBED_DIM, EMBED_DIM)] = row

        start_in(va, sia, wid)

        def pair_body(m, _):
            ta = wid + (2 * m) * NUM_WORKERS
            tb = ta + NUM_WORKERS
            ta2 = tb + NUM_WORKERS

            @pl.when(ta < nt)
            def _a():
                wait_in(va, sia, ta)

                @pl.when(tb < nt)
                def _pre_b():
                    start_in(vb, sib, tb)

                @pl.when(m > 0)
                def _drain_a():
                    wait_out(oa, soa, ta)

                transpose(va, oa)
                start_out(oa, soa, ta)

                @pl.when(tb < nt)
                def _b():
                    wait_in(vb, sib, tb)

                    @pl.when(ta2 < nt)
                    def _pre_a2():
                        start_in(va, sia, ta2)

                    @pl.when(m > 0)
                    def _drain_b():
                        wait_out(ob, sob, tb)

                    transpose(vb, ob)
                    start_out(ob, sob, tb)

            return 0

        lax.fori_loop(0, (TILES_PER_WORKER + 1) // 2, pair_body, 0)
        wait_out(oa, soa, 0)
        wait_out(ob, sob, 0)

    return detile


def _make_kernel():
    info = plsc.get_sparse_core_info()
    nc = info.num_cores
    mesh = plsc.VectorSubcoreMesh(core_axis_name="c", subcore_axis_name="s")

    @functools.partial(
        pl.kernel,
        out_type=jax.ShapeDtypeStruct((BATCH,), jnp.float32),
        mesh=mesh,
        scratch_types=[
            pltpu.VMEM((INPUT_SIZE, ROWS_PER_WORKER), jnp.int32),
            pltpu.VMEM((INPUT_SIZE, CHUNK_ROWS, EMBED_DIM), jnp.float32),
            pltpu.VMEM((INPUT_SIZE, CHUNK_ROWS, EMBED_DIM), jnp.float32),
            pltpu.VMEM((INPUT_SIZE * EMBED_DIM,), jnp.float32),
            pltpu.VMEM((EMBED_DIM,), jnp.float32),
            pltpu.VMEM((ROWS_PER_WORKER,), jnp.float32),
            pltpu.SemaphoreType.DMA,
            pltpu.SemaphoreType.DMA,
        ],
        compiler_params=pltpu.CompilerParams(
            needs_layout_passes=False, use_tc_tiling_on_sc=False),
    )
    def emb_kernel(x1_hbm, w_hbm, b_hbm, table_hbm, y_hbm,
                   idx_v, rows_a, rows_b, w_v, b_v, y_v, sem_a, sem_b):
        wid = lax.axis_index("s") * nc + lax.axis_index("c")
        row0 = wid * ROWS_PER_WORKER

        idx_descs = [
            pltpu.async_copy(
                x1_hbm.at[pl.ds(j * BATCH + row0, ROWS_PER_WORKER)],
                idx_v.at[j],
                sem_a,
            )
            for j in range(INPUT_SIZE)
        ]
        pltpu.sync_copy(w_hbm, w_v)
        pltpu.sync_copy(b_hbm, b_v)
        for d in idx_descs:
            d.wait()

        def gather(buf, c, sem):
            descs = []
            for j in range(INPUT_SIZE):
                descs.append(pltpu.async_copy(
                    table_hbm.at[idx_v.at[j, pl.ds(c * CHUNK_ROWS,
                                                   CHUNK_ROWS)]],
                    buf.at[j],
                    sem,
                ))
            return descs

        lane = lax.iota(jnp.int32, EMBED_DIM)

        def compute(buf, c):
            bias = b_v[:][0]

            def group_body(g, _):
                def row_body(rr, yvec):
                    r = g * 16 + rr
                    acc = buf[0, r, :] * w_v[pl.ds(0, EMBED_DIM)]
                    for j in range(1, INPUT_SIZE):
                        acc = (acc +
                               buf[j, r, :] * w_v[pl.ds(j * EMBED_DIM,
                                                        EMBED_DIM)])
                    val = jnp.sum(acc) + bias
                    return jnp.where(lane == rr, val, yvec)

                yvec = lax.fori_loop(
                    0, 16, row_body, jnp.zeros((EMBED_DIM,), jnp.float32))
                y_v[pl.ds(c * CHUNK_ROWS + g * 16, 16)] = yvec
                return 0

            lax.fori_loop(0, CHUNK_ROWS // 16, group_body, 0)

        bufs = (rows_a, rows_b)
        sems = (sem_a, sem_b)
        pending = gather(bufs[0], 0, sems[0])
        for c in range(NUM_CHUNKS):
            for d in pending:
                d.wait()
            if c + 1 < NUM_CHUNKS:
                pending = gather(bufs[(c + 1) % 2], c + 1, sems[(c + 1) % 2])
            compute(bufs[c % 2], c)

        pltpu.sync_copy(y_v, y_hbm.at[pl.ds(row0, ROWS_PER_WORKER)])

    return emb_kernel


_DETILE = _make_detile()
_EMB_KERNEL = _make_kernel()


@jax.jit
def kernel(X, table, W, b):
    tail = table[VOCAB - LAST_TILE_COLS:, :].reshape(-1)
    x1, tlin = _DETILE(X.T.astype(jnp.int32), table.T, tail)
    b16 = jnp.broadcast_to(b, (EMBED_DIM,)).astype(jnp.float32)
    y = _EMB_KERNEL(x1, W.reshape(-1), b16,
                    tlin.reshape(VOCAB, EMBED_DIM))
    return y.reshape(BATCH, 1)


# speedup vs baseline: 2.2609x; 1.5684x over previous
import functools
import jax
import jax.numpy as jnp
from jax import lax
from jax.experimental import pallas as pl
from jax.experimental.pallas import tpu as pltpu
from jax.experimental.pallas import tpu_sc as plsc

BATCH = 16384
INPUT_SIZE = 26
EMBED_DIM = 16
VOCAB = 1000000

NUM_WORKERS = 32
ROWS_PER_WORKER = BATCH // NUM_WORKERS
CHUNK_ROWS = 128
NUM_CHUNKS = ROWS_PER_WORKER // CHUNK_ROWS

LANE_TILES = (VOCAB + 127) // 128
TILES_PER_WORKER = (LANE_TILES + NUM_WORKERS - 1) // NUM_WORKERS
LAST_TILE_COLS = VOCAB - (LANE_TILES - 1) * 128


def _make_detile():
    info = plsc.get_sparse_core_info()
    nc = info.num_cores
    mesh = plsc.VectorSubcoreMesh(core_axis_name="c", subcore_axis_name="s")

    @functools.partial(
        pl.kernel,
        out_type=(
            jax.ShapeDtypeStruct((INPUT_SIZE * BATCH,), jnp.int32),
            jax.ShapeDtypeStruct((VOCAB * EMBED_DIM,), jnp.float32),
        ),
        mesh=mesh,
        scratch_types=[
            pltpu.VMEM((8, ROWS_PER_WORKER), jnp.int32),
            pltpu.VMEM((EMBED_DIM, 128), jnp.float32),
            pltpu.VMEM((EMBED_DIM, 128), jnp.float32),
            pltpu.VMEM((128 * EMBED_DIM,), jnp.float32),
            pltpu.VMEM((128 * EMBED_DIM,), jnp.float32),
            pltpu.SemaphoreType.DMA,
            pltpu.SemaphoreType.DMA,
            pltpu.SemaphoreType.DMA,
            pltpu.SemaphoreType.DMA,
        ],
        compiler_params=pltpu.CompilerParams(
            needs_layout_passes=False, use_tc_tiling_on_sc=True),
    )
    def detile(xt_hbm, tt_hbm, tail_hbm, xout_hbm, tout_hbm,
               xv, va, vb, oa, ob, sia, sib, soa, sob):
        wid = lax.axis_index("s") * nc + lax.axis_index("c")
        col0 = wid * ROWS_PER_WORKER

        @pl.when(wid == 0)
        def _tail():
            pltpu.sync_copy(tail_hbm, oa.at[pl.ds(0, LAST_TILE_COLS *
                                                  EMBED_DIM)])
            pltpu.sync_copy(
                oa.at[pl.ds(0, LAST_TILE_COLS * EMBED_DIM)],
                tout_hbm.at[pl.ds((LANE_TILES - 1) * 128 * EMBED_DIM,
                                  LAST_TILE_COLS * EMBED_DIM)])

        for t in range((INPUT_SIZE + 7) // 8):
            nr = min(8, INPUT_SIZE - t * 8)
            pltpu.sync_copy(
                xt_hbm.at[pl.ds(t * 8, nr), pl.ds(col0, ROWS_PER_WORKER)],
                xv.at[pl.ds(0, nr)])
            for r in range(nr):
                j = t * 8 + r
                pltpu.sync_copy(
                    xv.at[r],
                    xout_hbm.at[pl.ds(j * BATCH + col0, ROWS_PER_WORKER)])

        lane16 = lax.iota(jnp.int32, EMBED_DIM)
        nt = LANE_TILES - 1

        def start_in(v, sem, t):
            pltpu.async_copy(tt_hbm.at[:, pl.ds(t * 128, 128)], v, sem)

        def wait_in(v, sem, t):
            pltpu.make_async_copy(
                tt_hbm.at[:, pl.ds(t * 128, 128)], v, sem).wait()

        def start_out(o, sem, t):
            pltpu.async_copy(
                o, tout_hbm.at[pl.ds(t * 128 * EMBED_DIM,
                                     128 * EMBED_DIM)], sem)

        def wait_out(o, sem, t):
            pltpu.make_async_copy(
                o, tout_hbm.at[pl.ds(t * 128 * EMBED_DIM,
                                     128 * EMBED_DIM)], sem).wait()

        def transpose(v, o):
            for u0 in range(0, 128, 8):
                rows = [
                    plsc.load_gather(
                        v, [lane16, jnp.full((EMBED_DIM,), u0 + i, jnp.int32)])
                    for i in range(8)
                ]
                for i in range(8):
                    o[pl.ds((u0 + i) * EMBED_DIM, EMBED_DIM)] = rows[i]

        start_in(va, sia, wid)

        def pair_body(m, _):
            ta = wid + (2 * m) * NUM_WORKERS
            tb = ta + NUM_WORKERS
            ta2 = tb + NUM_WORKERS

            @pl.when(ta < nt)
            def _a():
                wait_in(va, sia, ta)

                @pl.when(tb < nt)
                def _pre_b():
                    start_in(vb, sib, tb)

                @pl.when(m > 0)
                def _drain_a():
                    wait_out(oa, soa, ta)

                transpose(va, oa)
                start_out(oa, soa, ta)

                @pl.when(tb < nt)
                def _b():
                    wait_in(vb, sib, tb)

                    @pl.when(ta2 < nt)
                    def _pre_a2():
                        start_in(va, sia, ta2)

                    @pl.when(m > 0)
                    def _drain_b():
                        wait_out(ob, sob, tb)

                    transpose(vb, ob)
                    start_out(ob, sob, tb)

            return 0

        lax.fori_loop(0, (TILES_PER_WORKER + 1) // 2, pair_body, 0)
        wait_out(oa, soa, 0)
        wait_out(ob, sob, 0)

    return detile


def _make_kernel():
    info = plsc.get_sparse_core_info()
    nc = info.num_cores
    mesh = plsc.VectorSubcoreMesh(core_axis_name="c", subcore_axis_name="s")

    @functools.partial(
        pl.kernel,
        out_type=jax.ShapeDtypeStruct((BATCH,), jnp.float32),
        mesh=mesh,
        scratch_types=[
            pltpu.VMEM((INPUT_SIZE, ROWS_PER_WORKER), jnp.int32),
            pltpu.VMEM((INPUT_SIZE, CHUNK_ROWS, EMBED_DIM), jnp.float32),
            pltpu.VMEM((INPUT_SIZE, CHUNK_ROWS, EMBED_DIM), jnp.float32),
            pltpu.VMEM((INPUT_SIZE * EMBED_DIM,), jnp.float32),
            pltpu.VMEM((EMBED_DIM,), jnp.float32),
            pltpu.VMEM((ROWS_PER_WORKER,), jnp.float32),
            pltpu.SemaphoreType.DMA,
            pltpu.SemaphoreType.DMA,
        ],
        compiler_params=pltpu.CompilerParams(
            needs_layout_passes=False, use_tc_tiling_on_sc=False),
    )
    def emb_kernel(x1_hbm, w_hbm, b_hbm, table_hbm, y_hbm,
                   idx_v, rows_a, rows_b, w_v, b_v, y_v, sem_a, sem_b):
        wid = lax.axis_index("s") * nc + lax.axis_index("c")
        row0 = wid * ROWS_PER_WORKER

        idx_descs = [
            pltpu.async_copy(
                x1_hbm.at[pl.ds(j * BATCH + row0, ROWS_PER_WORKER)],
                idx_v.at[j],
                sem_a,
            )
            for j in range(INPUT_SIZE)
        ]
        pltpu.sync_copy(w_hbm, w_v)
        pltpu.sync_copy(b_hbm, b_v)
        for d in idx_descs:
            d.wait()

        def gather(buf, c, sem):
            descs = []
            for j in range(INPUT_SIZE):
                descs.append(pltpu.async_copy(
                    table_hbm.at[idx_v.at[j, pl.ds(c * CHUNK_ROWS,
                                                   CHUNK_ROWS)]],
                    buf.at[j],
                    sem,
                ))
            return descs

        lane = lax.iota(jnp.int32, EMBED_DIM)

        def compute(buf, c):
            bias = b_v[:][0]

            def group_body(g, _):
                def row_body(rr, yvec):
                    r = g * 16 + rr
                    acc = buf[0, r, :] * w_v[pl.ds(0, EMBED_DIM)]
                    for j in range(1, INPUT_SIZE):
                        acc = (acc +
                               buf[j, r, :] * w_v[pl.ds(j * EMBED_DIM,
                                                        EMBED_DIM)])
                    val = jnp.sum(acc) + bias
                    return jnp.where(lane == rr, val, yvec)

                yvec = lax.fori_loop(
                    0, 16, row_body, jnp.zeros((EMBED_DIM,), jnp.float32))
                y_v[pl.ds(c * CHUNK_ROWS + g * 16, 16)] = yvec
                return 0

            lax.fori_loop(0, CHUNK_ROWS // 16, group_body, 0)

        bufs = (rows_a, rows_b)
        sems = (sem_a, sem_b)
        pending = gather(bufs[0], 0, sems[0])
        for c in range(NUM_CHUNKS):
            for d in pending:
                d.wait()
            if c + 1 < NUM_CHUNKS:
                pending = gather(bufs[(c + 1) % 2], c + 1, sems[(c + 1) % 2])
            compute(bufs[c % 2], c)

        pltpu.sync_copy(y_v, y_hbm.at[pl.ds(row0, ROWS_PER_WORKER)])

    return emb_kernel


_DETILE = _make_detile()
_EMB_KERNEL = _make_kernel()


@jax.jit
def kernel(X, table, W, b):
    tail = table[VOCAB - LAST_TILE_COLS:, :].reshape(-1)
    x1, tlin = _DETILE(X.T.astype(jnp.int32), table.T, tail)
    b16 = jnp.broadcast_to(b, (EMBED_DIM,)).astype(jnp.float32)
    y = _EMB_KERNEL(x1, W.reshape(-1), b16,
                    tlin.reshape(VOCAB, EMBED_DIM))
    return y.reshape(BATCH, 1)


# speedup vs baseline: 2.3014x; 1.0179x over previous
import functools
import jax
import jax.numpy as jnp
from jax import lax
from jax.experimental import pallas as pl
from jax.experimental.pallas import tpu as pltpu
from jax.experimental.pallas import tpu_sc as plsc

BATCH = 16384
INPUT_SIZE = 26
EMBED_DIM = 16
VOCAB = 1000000

NUM_WORKERS = 32
ROWS_PER_WORKER = BATCH // NUM_WORKERS
CHUNK_ROWS = 128
NUM_CHUNKS = ROWS_PER_WORKER // CHUNK_ROWS

LANE_TILES = (VOCAB + 127) // 128
LAST_TILE_COLS = VOCAB - (LANE_TILES - 1) * 128
SLAB_COLS = 256
NUM_SLABS = (LANE_TILES - 1) * 128 // SLAB_COLS
SLABS_PER_WORKER = (NUM_SLABS + NUM_WORKERS - 1) // NUM_WORKERS


def _make_detile():
    info = plsc.get_sparse_core_info()
    nc = info.num_cores
    mesh = plsc.VectorSubcoreMesh(core_axis_name="c", subcore_axis_name="s")

    @functools.partial(
        pl.kernel,
        out_type=(
            jax.ShapeDtypeStruct((INPUT_SIZE * BATCH,), jnp.int32),
            jax.ShapeDtypeStruct((VOCAB * EMBED_DIM,), jnp.float32),
        ),
        mesh=mesh,
        scratch_types=[
            pltpu.VMEM((8, ROWS_PER_WORKER), jnp.int32),
            pltpu.VMEM((EMBED_DIM, SLAB_COLS), jnp.float32),
            pltpu.VMEM((EMBED_DIM, SLAB_COLS), jnp.float32),
            pltpu.VMEM((SLAB_COLS * EMBED_DIM,), jnp.float32),
            pltpu.VMEM((SLAB_COLS * EMBED_DIM,), jnp.float32),
            pltpu.SemaphoreType.DMA,
            pltpu.SemaphoreType.DMA,
            pltpu.SemaphoreType.DMA,
            pltpu.SemaphoreType.DMA,
        ],
        compiler_params=pltpu.CompilerParams(
            needs_layout_passes=False, use_tc_tiling_on_sc=True),
    )
    def detile(xt_hbm, tt_hbm, tail_hbm, xout_hbm, tout_hbm,
               xv, va, vb, oa, ob, sia, sib, soa, sob):
        wid = lax.axis_index("s") * nc + lax.axis_index("c")
        col0 = wid * ROWS_PER_WORKER

        @pl.when(wid == 0)
        def _tail():
            pltpu.sync_copy(tail_hbm, oa.at[pl.ds(0, LAST_TILE_COLS *
                                                  EMBED_DIM)])
            pltpu.sync_copy(
                oa.at[pl.ds(0, LAST_TILE_COLS * EMBED_DIM)],
                tout_hbm.at[pl.ds((LANE_TILES - 1) * 128 * EMBED_DIM,
                                  LAST_TILE_COLS * EMBED_DIM)])

        for t in range((INPUT_SIZE + 7) // 8):
            nr = min(8, INPUT_SIZE - t * 8)
            pltpu.sync_copy(
                xt_hbm.at[pl.ds(t * 8, nr), pl.ds(col0, ROWS_PER_WORKER)],
                xv.at[pl.ds(0, nr)])
            for r in range(nr):
                j = t * 8 + r
                pltpu.sync_copy(
                    xv.at[r],
                    xout_hbm.at[pl.ds(j * BATCH + col0, ROWS_PER_WORKER)])

        lane16 = lax.iota(jnp.int32, EMBED_DIM)
        nt = NUM_SLABS

        def start_in(v, sem, t):
            pltpu.async_copy(tt_hbm.at[:, pl.ds(t * SLAB_COLS, SLAB_COLS)],
                             v, sem)

        def wait_in(v, sem, t):
            pltpu.make_async_copy(
                tt_hbm.at[:, pl.ds(t * SLAB_COLS, SLAB_COLS)], v, sem).wait()

        def start_out(o, sem, t):
            pltpu.async_copy(
                o, tout_hbm.at[pl.ds(t * SLAB_COLS * EMBED_DIM,
                                     SLAB_COLS * EMBED_DIM)], sem)

        def wait_out(o, sem, t):
            pltpu.make_async_copy(
                o, tout_hbm.at[pl.ds(t * SLAB_COLS * EMBED_DIM,
                                     SLAB_COLS * EMBED_DIM)], sem).wait()

        def transpose(v, o):
            def grp(g, _):
                base = g * 16
                rows = [
                    plsc.load_gather(
                        v, [lane16,
                            jnp.full((EMBED_DIM,), base + i, jnp.int32)])
                    for i in range(16)
                ]
                for i in range(16):
                    o[pl.ds((base + i) * EMBED_DIM, EMBED_DIM)] = rows[i]
                return 0

            lax.fori_loop(0, SLAB_COLS // 16, grp, 0)

        start_in(va, sia, wid)

        def pair_body(m, _):
            ta = wid + (2 * m) * NUM_WORKERS
            tb = ta + NUM_WORKERS
            ta2 = tb + NUM_WORKERS

            @pl.when(ta < nt)
            def _a():
                wait_in(va, sia, ta)

                @pl.when(tb < nt)
                def _pre_b():
                    start_in(vb, sib, tb)

                @pl.when(m > 0)
                def _drain_a():
                    wait_out(oa, soa, ta)

                transpose(va, oa)
                start_out(oa, soa, ta)

                @pl.when(tb < nt)
                def _b():
                    wait_in(vb, sib, tb)

                    @pl.when(ta2 < nt)
                    def _pre_a2():
                        start_in(va, sia, ta2)

                    @pl.when(m > 0)
                    def _drain_b():
                        wait_out(ob, sob, tb)

                    transpose(vb, ob)
                    start_out(ob, sob, tb)

            return 0

        lax.fori_loop(0, (SLABS_PER_WORKER + 1) // 2, pair_body, 0)
        wait_out(oa, soa, 0)
        wait_out(ob, sob, 0)

    return detile


def _make_kernel():
    info = plsc.get_sparse_core_info()
    nc = info.num_cores
    mesh = plsc.VectorSubcoreMesh(core_axis_name="c", subcore_axis_name="s")

    @functools.partial(
        pl.kernel,
        out_type=jax.ShapeDtypeStruct((BATCH,), jnp.float32),
        mesh=mesh,
        scratch_types=[
            pltpu.VMEM((INPUT_SIZE, ROWS_PER_WORKER), jnp.int32),
            pltpu.VMEM((INPUT_SIZE, CHUNK_ROWS, EMBED_DIM), jnp.float32),
            pltpu.VMEM((INPUT_SIZE, CHUNK_ROWS, EMBED_DIM), jnp.float32),
            pltpu.VMEM((INPUT_SIZE * EMBED_DIM,), jnp.float32),
            pltpu.VMEM((EMBED_DIM,), jnp.float32),
            pltpu.VMEM((ROWS_PER_WORKER,), jnp.float32),
            pltpu.SemaphoreType.DMA,
            pltpu.SemaphoreType.DMA,
        ],
        compiler_params=pltpu.CompilerParams(
            needs_layout_passes=False, use_tc_tiling_on_sc=False),
    )
    def emb_kernel(x1_hbm, w_hbm, b_hbm, table_hbm, y_hbm,
                   idx_v, rows_a, rows_b, w_v, b_v, y_v, sem_a, sem_b):
        wid = lax.axis_index("s") * nc + lax.axis_index("c")
        row0 = wid * ROWS_PER_WORKER

        idx_descs = [
            pltpu.async_copy(
                x1_hbm.at[pl.ds(j * BATCH + row0, ROWS_PER_WORKER)],
                idx_v.at[j],
                sem_a,
            )
            for j in range(INPUT_SIZE)
        ]
        pltpu.sync_copy(w_hbm, w_v)
        pltpu.sync_copy(b_hbm, b_v)
        for d in idx_descs:
            d.wait()

        def gather(buf, c, sem):
            descs = []
            for j in range(INPUT_SIZE):
                descs.append(pltpu.async_copy(
                    table_hbm.at[idx_v.at[j, pl.ds(c * CHUNK_ROWS,
                                                   CHUNK_ROWS)]],
                    buf.at[j],
                    sem,
                ))
            return descs

        lane = lax.iota(jnp.int32, EMBED_DIM)

        def compute(buf, c):
            bias = b_v[:][0]

            def group_body(g, _):
                def row_body(rr, yvec):
                    r = g * 16 + rr
                    acc = buf[0, r, :] * w_v[pl.ds(0, EMBED_DIM)]
                    for j in range(1, INPUT_SIZE):
                        acc = (acc +
                               buf[j, r, :] * w_v[pl.ds(j * EMBED_DIM,
                                                        EMBED_DIM)])
                    val = jnp.sum(acc) + bias
                    return jnp.where(lane == rr, val, yvec)

                yvec = lax.fori_loop(
                    0, 16, row_body, jnp.zeros((EMBED_DIM,), jnp.float32))
                y_v[pl.ds(c * CHUNK_ROWS + g * 16, 16)] = yvec
                return 0

            lax.fori_loop(0, CHUNK_ROWS // 16, group_body, 0)

        bufs = (rows_a, rows_b)
        sems = (sem_a, sem_b)
        pending = gather(bufs[0], 0, sems[0])
        for c in range(NUM_CHUNKS):
            for d in pending:
                d.wait()
            if c + 1 < NUM_CHUNKS:
                pending = gather(bufs[(c + 1) % 2], c + 1, sems[(c + 1) % 2])
            compute(bufs[c % 2], c)

        pltpu.sync_copy(y_v, y_hbm.at[pl.ds(row0, ROWS_PER_WORKER)])

    return emb_kernel


_DETILE = _make_detile()
_EMB_KERNEL = _make_kernel()


@jax.jit
def kernel(X, table, W, b):
    tail = table[VOCAB - LAST_TILE_COLS:, :].reshape(-1)
    x1, tlin = _DETILE(X.T.astype(jnp.int32), table.T, tail)
    b16 = jnp.broadcast_to(b, (EMBED_DIM,)).astype(jnp.float32)
    y = _EMB_KERNEL(x1, W.reshape(-1), b16,
                    tlin.reshape(VOCAB, EMBED_DIM))
    return y.reshape(BATCH, 1)
